# baseline (speedup 1.0000x reference)
import jax
import jax.numpy as jnp
from jax import lax
from jax.experimental import pallas as pl
from jax.experimental.pallas import tpu as pltpu
from jax.experimental.pallas import tpu_sc as plsc

BASE_SCALE = 150.0
BASE_SHIFT = 10.0
H, W = 2160, 3840
NW = 32
NROWS = (H + NW - 1) // NW
NV = W // 16


def _sc_body(raw_hbm, coeff_hbm, out_hbm, coeff_v, u_v, row_in, p0, p1, p2):
    wid = lax.axis_index("s") * 2 + lax.axis_index("c")

    pltpu.sync_copy(coeff_hbm, coeff_v)

    def build_u(j, _):
        u_v[pl.ds(j * 16, 16)] = (lax.iota(jnp.int32, 16) + j * 16).astype(
            jnp.float32)
        return 0

    lax.fori_loop(0, NV, build_u, 0, unroll=4)

    A = coeff_v[0, :]
    B = coeff_v[1, :]
    c0 = [coeff_v[2 + k, :] for k in range(3)]
    c1 = [coeff_v[5 + k, :] for k in range(3)]
    c2 = [coeff_v[8 + k, :] for k in range(3)]
    t = [coeff_v[11 + k, :] for k in range(3)]

    def row_step(r, _):
        row = wid + NW * r

        @pl.when(row < H)
        def _():
            pltpu.sync_copy(raw_hbm.at[row], row_in)
            vf = jnp.full((16,), row, dtype=jnp.int32).astype(jnp.float32)
            h = [vf * c1[k] + c2[k] for k in range(3)]
            outs = [p0, p1, p2]

            def col_step(j, _):
                sl = pl.ds(j * 16, 16)
                d = row_in[sl] * A + B
                uv = u_v[sl]
                for k in range(3):
                    outs[k][sl] = d * (uv * c0[k] + h[k]) + t[k]
                return 0

            lax.fori_loop(0, NV, col_step, 0, unroll=2)
            for k in range(3):
                pltpu.sync_copy(outs[k], out_hbm.at[k, row])

        return 0

    lax.fori_loop(0, NROWS, row_step, 0)


def kernel(raw_depth, quaternion, T, scale, shift, Focalx, Focaly, Offsetx, Offsety):
    q = quaternion / jnp.sqrt(jnp.sum(quaternion * quaternion) + 1e-12)
    w, x, y, z = q[0], q[1], q[2], q[3]
    R = jnp.stack([
        jnp.stack([1 - 2 * (y * y + z * z), 2 * (x * y - w * z), 2 * (x * z + w * y)]),
        jnp.stack([2 * (x * y + w * z), 1 - 2 * (x * x + z * z), 2 * (y * z - w * x)]),
        jnp.stack([2 * (x * z - w * y), 2 * (y * z + w * x), 1 - 2 * (x * x + y * y)]),
    ])
    A = jnp.exp(scale) * BASE_SCALE
    B = shift * A + BASE_SHIFT
    c0 = R[:, 0] / Focalx
    c1 = R[:, 1] / Focaly
    c2 = R[:, 2] - R[:, 0] * Offsetx / Focalx - R[:, 1] * Offsety / Focaly
    coeffs = jnp.concatenate([
        jnp.stack([A, B]), c0, c1, c2, T, jnp.zeros((2,), jnp.float32)
    ]).astype(jnp.float32)
    coeff_tab = jnp.broadcast_to(coeffs[:, None], (16, 16))

    mesh = plsc.VectorSubcoreMesh(core_axis_name="c", subcore_axis_name="s")
    sc_call = pl.kernel(
        _sc_body,
        mesh=mesh,
        out_type=jax.ShapeDtypeStruct((3, H, W), jnp.float32),
        scratch_types=[
            pltpu.VMEM((16, 16), jnp.float32),
            pltpu.VMEM((W,), jnp.float32),
            pltpu.VMEM((W,), jnp.float32),
            pltpu.VMEM((W,), jnp.float32),
            pltpu.VMEM((W,), jnp.float32),
            pltpu.VMEM((W,), jnp.float32),
        ],
    )
    planes = sc_call(raw_depth, coeff_tab)
    return jnp.transpose(planes, (1, 2, 0))


# speedup vs baseline: 6.0734x; 6.0734x over previous
import jax
import jax.numpy as jnp
from jax import lax
from jax.experimental import pallas as pl
from jax.experimental.pallas import tpu as pltpu

BASE_SCALE = 150.0
BASE_SHIFT = 10.0
H, W = 2160, 3840
BH = 240


def _body(params_ref, raw_ref, out_ref):
    p = params_ref
    q0, q1, q2, q3 = p[0], p[1], p[2], p[3]
    t0, t1, t2 = p[4], p[5], p[6]
    scale, shift = p[7], p[8]
    fx, fy, ox, oy = p[9], p[10], p[11], p[12]

    inv = lax.rsqrt(q0 * q0 + q1 * q1 + q2 * q2 + q3 * q3 + 1e-12)
    w = q0 * inv
    x = q1 * inv
    y = q2 * inv
    z = q3 * inv
    r00 = 1 - 2 * (y * y + z * z)
    r01 = 2 * (x * y - w * z)
    r02 = 2 * (x * z + w * y)
    r10 = 2 * (x * y + w * z)
    r11 = 1 - 2 * (x * x + z * z)
    r12 = 2 * (y * z - w * x)
    r20 = 2 * (x * z - w * y)
    r21 = 2 * (y * z + w * x)
    r22 = 1 - 2 * (x * x + y * y)

    es = jnp.exp(scale) * BASE_SCALE
    A = es
    B = shift * es + BASE_SHIFT

    i = pl.program_id(0)
    u1 = lax.broadcasted_iota(jnp.int32, (1, W), 1).astype(jnp.float32)
    v1 = (lax.broadcasted_iota(jnp.int32, (BH, 1), 0) + i * BH).astype(jnp.float32)

    d = raw_ref[...] * A + B

    ifx = 1.0 / fx
    ify = 1.0 / fy

    def plane(ra, rb, rc, t):
        c0 = ra * ifx
        c1 = rb * ify
        c2 = rc - ra * ox * ifx - rb * oy * ify
        a = (u1 * c0 + c2) + v1 * c1
        return d * a + t

    out_ref[0, :, :] = plane(r00, r01, r02, t0)
    out_ref[1, :, :] = plane(r10, r11, r12, t1)
    out_ref[2, :, :] = plane(r20, r21, r22, t2)


def kernel(raw_depth, quaternion, T, scale, shift, Focalx, Focaly, Offsetx, Offsety):
    params = jnp.concatenate([
        quaternion,
        T,
        jnp.stack([scale, shift, Focalx, Focaly, Offsetx, Offsety]),
    ]).astype(jnp.float32)

    planes = pl.pallas_call(
        _body,
        grid=(H // BH,),
        in_specs=[
            pl.BlockSpec(memory_space=pltpu.SMEM),
            pl.BlockSpec((BH, W), lambda i: (i, 0)),
        ],
        out_specs=pl.BlockSpec((3, BH, W), lambda i: (0, i, 0)),
        out_shape=jax.ShapeDtypeStruct((3, H, W), jnp.float32),
        compiler_params=pltpu.CompilerParams(
            dimension_semantics=("parallel",),
        ),
    )(params, raw_depth)
    return jnp.transpose(planes, (1, 2, 0))
